# baseline (speedup 1.0000x reference)
import functools

import jax
import jax.numpy as jnp
from jax import lax
from jax.experimental import pallas as pl
from jax.experimental.pallas import tpu as pltpu
from jax.experimental.pallas import tpu_sc as plsc

_N = 100000
_D = 64
_L = 128
_NW = 32
_CHUNKS = 24
_NBUF = 4
_BPW = _CHUNKS * _L
_MAIN = _NW * _BPW
_EXTRA = 13
_TAIL = _N - _MAIN - _EXTRA * _L


def _make_gather():
    mesh = plsc.VectorSubcoreMesh(core_axis_name="c", subcore_axis_name="s")

    @functools.partial(
        pl.kernel,
        mesh=mesh,
        out_type=jax.ShapeDtypeStruct((_N, 2 * _D), jnp.float32),
        compiler_params=pltpu.CompilerParams(
            use_tc_tiling_on_sc=True, needs_layout_passes=False,
            skip_device_barrier=True),
        scratch_types=(
            [pltpu.VMEM((_BPW,), jnp.int32),
             pltpu.VMEM((_NBUF, _L, 2 * _D), jnp.float32),
             pltpu.VMEM((_L,), jnp.int32),
             pltpu.VMEM((_TAIL,), jnp.int32),
             pltpu.VMEM((_TAIL, 2 * _D), jnp.float32)]
            + [pltpu.SemaphoreType.DMA] * (2 * _NBUF + 1)
        ),
    )
    def gather_kernel(table_hbm, idx_hbm, out_hbm,
                      idx_v, rows_v, xidx_v, tidx_v, trows_v, *sems):
        gsem, osem, xsem = sems[:_NBUF], sems[_NBUF:2 * _NBUF], sems[-1]
        wid = lax.axis_index("s") * 2 + lax.axis_index("c")
        base = wid * _BPW
        pltpu.sync_copy(idx_hbm.at[pl.ds(base, _BPW)], idx_v)
        gathers = [None] * _NBUF
        outs = [None] * _NBUF
        for g in range(_CHUNKS + _NBUF - 1):
            if g < _CHUNKS:
                b = g % _NBUF
                if g >= _NBUF:
                    outs[b].wait()
                gathers[b] = pltpu.async_copy(
                    table_hbm.at[idx_v.at[pl.ds(g * _L, _L)]],
                    rows_v.at[b], gsem[b])
            d = g - (_NBUF - 1)
            if d >= 0:
                db = d % _NBUF
                gathers[db].wait()
                outs[db] = pltpu.async_copy(
                    rows_v.at[db], out_hbm.at[pl.ds(base + d * _L, _L)],
                    osem[db])
        for d in range(max(0, _CHUNKS - _NBUF), _CHUNKS):
            outs[d % _NBUF].wait()

        @pl.when(wid < _EXTRA)
        def _extra():
            start = _MAIN + wid * _L
            pltpu.sync_copy(idx_hbm.at[pl.ds(start, _L)], xidx_v)
            pltpu.async_copy(table_hbm.at[xidx_v], rows_v.at[0], xsem).wait()
            pltpu.sync_copy(rows_v.at[0], out_hbm.at[pl.ds(start, _L)])

        @pl.when(wid == _EXTRA)
        def _tail():
            start = _MAIN + _EXTRA * _L
            pltpu.sync_copy(idx_hbm.at[pl.ds(start, _TAIL)], tidx_v)
            pltpu.async_copy(table_hbm.at[tidx_v], trows_v, xsem).wait()
            pltpu.sync_copy(trows_v, out_hbm.at[pl.ds(start, _TAIL)])

    return gather_kernel


_gather = _make_gather()


def kernel(node_ids, x, etypes, norm, table):
    del x, etypes, norm
    table2 = jnp.concatenate([table, table], axis=1)
    out2 = _gather(table2, node_ids.astype(jnp.int32))
    return out2[:, :_D]


# speedup vs baseline: 1.0935x; 1.0935x over previous
import functools

import jax
import jax.numpy as jnp
from jax import lax
from jax.experimental import pallas as pl
from jax.experimental.pallas import tpu as pltpu
from jax.experimental.pallas import tpu_sc as plsc

_N = 100000
_D = 64
_L = 128
_NW = 32
_CHUNKS = 24
_NBUF = 8
_BPW = _CHUNKS * _L
_MAIN = _NW * _BPW
_EXTRA = 13
_TAIL = _N - _MAIN - _EXTRA * _L


def _make_gather():
    mesh = plsc.VectorSubcoreMesh(core_axis_name="c", subcore_axis_name="s")

    @functools.partial(
        pl.kernel,
        mesh=mesh,
        out_type=jax.ShapeDtypeStruct((_N, _D), jnp.float32),
        compiler_params=pltpu.CompilerParams(
            use_tc_tiling_on_sc=False, skip_device_barrier=True),
        scratch_types=(
            [pltpu.VMEM((_BPW,), jnp.int32),
             pltpu.VMEM((_NBUF, _L, _D), jnp.float32),
             pltpu.VMEM((_L,), jnp.int32),
             pltpu.VMEM((_TAIL,), jnp.int32),
             pltpu.VMEM((_TAIL, _D), jnp.float32)]
            + [pltpu.SemaphoreType.DMA] * (2 * _NBUF + 1)
        ),
    )
    def gather_kernel(table_hbm, idx_hbm, out_hbm,
                      idx_v, rows_v, xidx_v, tidx_v, trows_v, *sems):
        gsem, osem, xsem = sems[:_NBUF], sems[_NBUF:2 * _NBUF], sems[-1]
        wid = lax.axis_index("s") * 2 + lax.axis_index("c")
        base = wid * _BPW
        pltpu.sync_copy(idx_hbm.at[pl.ds(base, _BPW)], idx_v)
        gathers = [None] * _NBUF
        outs = [None] * _NBUF
        for g in range(_CHUNKS + _NBUF - 1):
            if g < _CHUNKS:
                b = g % _NBUF
                if g >= _NBUF:
                    outs[b].wait()
                gathers[b] = pltpu.async_copy(
                    table_hbm.at[idx_v.at[pl.ds(g * _L, _L)]],
                    rows_v.at[b], gsem[b])
            d = g - (_NBUF - 1)
            if d >= 0:
                db = d % _NBUF
                gathers[db].wait()
                outs[db] = pltpu.async_copy(
                    rows_v.at[db], out_hbm.at[pl.ds(base + d * _L, _L)],
                    osem[db])
        for d in range(max(0, _CHUNKS - _NBUF), _CHUNKS):
            outs[d % _NBUF].wait()

        @pl.when(wid < _EXTRA)
        def _extra():
            start = _MAIN + wid * _L
            pltpu.sync_copy(idx_hbm.at[pl.ds(start, _L)], xidx_v)
            pltpu.async_copy(table_hbm.at[xidx_v], rows_v.at[0], xsem).wait()
            pltpu.sync_copy(rows_v.at[0], out_hbm.at[pl.ds(start, _L)])

        @pl.when(wid == _EXTRA)
        def _tail():
            start = _MAIN + _EXTRA * _L
            pltpu.sync_copy(idx_hbm.at[pl.ds(start, _TAIL)], tidx_v)
            pltpu.async_copy(table_hbm.at[tidx_v], trows_v, xsem).wait()
            pltpu.sync_copy(trows_v, out_hbm.at[pl.ds(start, _TAIL)])

    return gather_kernel


_gather = _make_gather()


def kernel(node_ids, x, etypes, norm, table):
    del x, etypes, norm
    return _gather(table, node_ids.astype(jnp.int32))


# speedup vs baseline: 1.1874x; 1.0859x over previous
import functools

import jax
import jax.numpy as jnp
from jax import lax
from jax.experimental import pallas as pl
from jax.experimental.pallas import tpu as pltpu
from jax.experimental.pallas import tpu_sc as plsc

_N = 100000
_D = 64
_L = 128
_NW = 32
_CHUNKS = 24
_NBUF = 6
_BPW = _CHUNKS * _L
_MAIN = _NW * _BPW
_EXTRA = 13
_TAIL = _N - _MAIN - _EXTRA * _L


def _make_gather():
    mesh = plsc.VectorSubcoreMesh(core_axis_name="c", subcore_axis_name="s")

    @functools.partial(
        pl.kernel,
        mesh=mesh,
        out_type=jax.ShapeDtypeStruct((_N, _D), jnp.float32),
        compiler_params=pltpu.CompilerParams(
            use_tc_tiling_on_sc=False, skip_device_barrier=True),
        scratch_types=(
            [pltpu.VMEM((_BPW,), jnp.int32),
             pltpu.VMEM((_NBUF, _L, 2 * _D), jnp.float32),
             pltpu.VMEM((_L,), jnp.int32),
             pltpu.VMEM((_TAIL,), jnp.int32),
             pltpu.VMEM((_TAIL, 2 * _D), jnp.float32)]
            + [pltpu.SemaphoreType.DMA] * (2 * _NBUF + 1)
        ),
    )
    def gather_kernel(table_hbm, idx_hbm, out_hbm,
                      idx_v, rows_v, xidx_v, tidx_v, trows_v, *sems):
        gsem, osem, xsem = sems[:_NBUF], sems[_NBUF:2 * _NBUF], sems[-1]
        wid = lax.axis_index("s") * 2 + lax.axis_index("c")
        base = wid * _BPW
        pltpu.sync_copy(idx_hbm.at[pl.ds(base, _BPW)], idx_v)
        gathers = [None] * _NBUF
        outs = [None] * _NBUF
        for g in range(_CHUNKS + _NBUF - 1):
            if g < _CHUNKS:
                b = g % _NBUF
                if g >= _NBUF:
                    outs[b].wait()
                gathers[b] = pltpu.async_copy(
                    table_hbm.at[idx_v.at[pl.ds(g * _L, _L)]],
                    rows_v.at[b], gsem[b])
            d = g - (_NBUF - 1)
            if d >= 0:
                db = d % _NBUF
                gathers[db].wait()
                outs[db] = pltpu.async_copy(
                    rows_v.at[db, :, pl.ds(0, _D)],
                    out_hbm.at[pl.ds(base + d * _L, _L)], osem[db])
        for d in range(max(0, _CHUNKS - _NBUF), _CHUNKS):
            outs[d % _NBUF].wait()

        @pl.when(wid < _EXTRA)
        def _extra():
            start = _MAIN + wid * _L
            pltpu.sync_copy(idx_hbm.at[pl.ds(start, _L)], xidx_v)
            pltpu.async_copy(table_hbm.at[xidx_v], rows_v.at[0], xsem).wait()
            pltpu.sync_copy(rows_v.at[0, :, pl.ds(0, _D)],
                            out_hbm.at[pl.ds(start, _L)])

        @pl.when(wid == _EXTRA)
        def _tail():
            start = _MAIN + _EXTRA * _L
            pltpu.sync_copy(idx_hbm.at[pl.ds(start, _TAIL)], tidx_v)
            pltpu.async_copy(table_hbm.at[tidx_v], trows_v, xsem).wait()
            pltpu.sync_copy(trows_v.at[:, pl.ds(0, _D)],
                            out_hbm.at[pl.ds(start, _TAIL)])

    return gather_kernel


_gather = _make_gather()


def kernel(node_ids, x, etypes, norm, table):
    del x, etypes, norm
    table2 = jnp.pad(table, ((0, 0), (0, _D)))
    return _gather(table2, node_ids.astype(jnp.int32))
